# Initial kernel scaffold; baseline (speedup 1.0000x reference)
import jax
import jax.numpy as jnp
from jax.experimental import pallas as pl


def kernel(hidden_feat, node_feat, edge_index, sample_weights, q_probs, W, b):
    raise NotImplementedError("write your pallas kernel here")


# speedup vs baseline: 25.5650x; 25.5650x over previous
import functools

import jax
import jax.numpy as jnp
from jax import lax
from jax.experimental import pallas as pl
from jax.experimental.pallas import tpu as pltpu
from jax.experimental.pallas import tpu_sc as plsc

N = 10000
E = 320000
D = 128

NC = 2
NS = 16
NW = NC * NS
EPW = E // NW
C = 80
NCHUNK = EPW // C
OWN = 1000
ZR = 250

_mesh = plsc.VectorSubcoreMesh(
    core_axis_name="c", subcore_axis_name="s", num_cores=NC, num_subcores=NS)
_sc_params = pltpu.CompilerParams(needs_layout_passes=False)
_sc_params_deg = pltpu.CompilerParams(needs_layout_passes=False,
                                      use_tc_tiling_on_sc=False)


@functools.partial(
    pl.kernel,
    out_type=jax.ShapeDtypeStruct((NC * N, 16), jnp.float32),
    mesh=_mesh,
    scratch_types=[
        pltpu.VMEM((C,), jnp.int32),
        pltpu.VMEM((C, 16), jnp.float32),
        pltpu.VMEM((ZR, 16), jnp.float32),
        pltpu.VMEM_SHARED((N, 16), jnp.float32),
    ],
    compiler_params=_sc_params_deg,
)
def _sc_degree(dst_hbm, deg_out, didx_v, ones_v, zbuf_v, acc_sh):
    cid = lax.axis_index("c")
    sid = lax.axis_index("s")
    wid = sid * NC + cid

    def ones_body(r, _):
        ones_v[r, :] = jnp.ones((16,), jnp.float32)
        return 0
    lax.fori_loop(0, C, ones_body, 0)

    def zfill_body(r, _):
        zbuf_v[r, :] = jnp.zeros((16,), jnp.float32)
        return 0
    lax.fori_loop(0, ZR, zfill_body, 0)

    @pl.when(sid < N // OWN)
    def _():
        def zero_body(k, _):
            pltpu.sync_copy(zbuf_v,
                            acc_sh.at[pl.ds(sid * OWN + k * ZR, ZR)])
            return 0
        lax.fori_loop(0, OWN // ZR, zero_body, 0)
    plsc.subcore_barrier()

    def chunk_body(i, _):
        base = wid * EPW + i * C
        pltpu.sync_copy(dst_hbm.at[pl.ds(base, C)], didx_v)
        pltpu.sync_copy(ones_v, acc_sh.at[didx_v], add=True)
        return 0
    lax.fori_loop(0, NCHUNK, chunk_body, 0)
    plsc.subcore_barrier()

    @pl.when(sid < N // OWN)
    def _():
        pltpu.sync_copy(acc_sh.at[pl.ds(sid * OWN, OWN)],
                        deg_out.at[pl.ds(cid * N + sid * OWN, OWN)])


def _tc_scalars_body(nf_ref, hid_ref, sw_ref, q_ref, degp_ref,
                     hu_ref, hv_ref, nd_ref, hp_ref):
    deg = degp_ref[0:N, 0:1] + degp_ref[N:2 * N, 0:1]
    nd = lax.rsqrt(deg + 1.0)
    nd_ref[...] = nd
    c = nd / (q_ref[...] * float(E))
    hp_ref[...] = hid_ref[...] * c
    nf = nf_ref[...]
    hu_ref[...] = jnp.dot(nf, sw_ref[:, 0:1],
                          preferred_element_type=jnp.float32)
    hv_ref[...] = jnp.dot(nf, sw_ref[:, 1:2],
                          preferred_element_type=jnp.float32)


def _tc_scalars(node_feat, hidden_feat, sample_weights, q2, deg_part):
    out = jax.ShapeDtypeStruct((N, 1), jnp.float32)
    return pl.pallas_call(
        _tc_scalars_body,
        out_shape=(out, out, out, jax.ShapeDtypeStruct((N, D), jnp.float32)),
    )(node_feat, hidden_feat, sample_weights, q2, deg_part)


@functools.partial(
    pl.kernel,
    out_type=jax.ShapeDtypeStruct((NC * N, D), jnp.float32),
    mesh=_mesh,
    scratch_types=[
        pltpu.VMEM((C,), jnp.int32),
        pltpu.VMEM((C,), jnp.int32),
        pltpu.VMEM((C, D), jnp.float32),
        pltpu.VMEM((N,), jnp.float32),
        pltpu.VMEM((N,), jnp.float32),
        pltpu.SemaphoreType.DMA,
        pltpu.VMEM_SHARED((N, D), jnp.float32),
    ],
    compiler_params=_sc_params,
)
def _sc_edges(hp_hbm, src_hbm, dst_hbm, hu_hbm, hv_hbm, zeros_hbm, part_out,
              sidx_v, didx_v, rows_v, hu_v, hv_v,
              sem, acc_sh):
    cid = lax.axis_index("c")
    sid = lax.axis_index("s")
    wid = sid * NC + cid

    pltpu.sync_copy(hu_hbm, hu_v)
    pltpu.sync_copy(hv_hbm, hv_v)

    @pl.when(sid < N // OWN)
    def _():
        def zero_body(k, _):
            pltpu.sync_copy(zeros_hbm,
                            acc_sh.at[pl.ds(sid * OWN + k * ZR, ZR)])
            return 0
        lax.fori_loop(0, OWN // ZR, zero_body, 0)
    plsc.subcore_barrier()

    def chunk_body(i, _):
        base = wid * EPW + i * C
        pltpu.sync_copy(src_hbm.at[pl.ds(base, C)], sidx_v)
        pltpu.sync_copy(dst_hbm.at[pl.ds(base, C)], didx_v)
        pltpu.async_copy(hp_hbm.at[sidx_v], rows_v, sem).wait()

        def grp_body(g, _):
            s16 = pl.ds(g * 16, 16)
            si = sidx_v[s16]
            di = didx_v[s16]
            hu16 = plsc.load_gather(hu_v, [si])
            hv16 = plsc.load_gather(hv_v, [di])
            s = jnp.maximum(hu16 + hv16, 0.0) + 0.1
            for e in range(16):
                a = s[e]
                row = g * 16 + e
                for j in range(D // 16):
                    sl = pl.ds(j * 16, 16)
                    rows_v[row, sl] = rows_v[row, sl] * a
            return 0
        lax.fori_loop(0, C // 16, grp_body, 0)

        pltpu.sync_copy(rows_v, acc_sh.at[didx_v], add=True)
        return 0
    lax.fori_loop(0, NCHUNK, chunk_body, 0)
    plsc.subcore_barrier()

    @pl.when(sid < N // OWN)
    def _():
        pltpu.sync_copy(acc_sh.at[pl.ds(sid * OWN, OWN)],
                        part_out.at[pl.ds(cid * N + sid * OWN, OWN)])


def _tc_final_body(part_ref, nd_ref, w_ref, b_ref, out_ref):
    neigh = (part_ref[0:N, :] + part_ref[N:2 * N, :]) * nd_ref[...]
    out_ref[...] = lax.dot_general(
        neigh, w_ref[...], (((1,), (1,)), ((), ())),
        preferred_element_type=jnp.float32) + b_ref[...]


def _tc_final(part, nd2, W, b2):
    return pl.pallas_call(
        _tc_final_body,
        out_shape=jax.ShapeDtypeStruct((N, D), jnp.float32),
    )(part, nd2, W, b2)


def kernel(hidden_feat, node_feat, edge_index, sample_weights, q_probs, W, b):
    src = edge_index[0].astype(jnp.int32)
    dst = edge_index[1].astype(jnp.int32)

    deg_part = _sc_degree(dst)
    hu, hv, nd, hprime = _tc_scalars(node_feat, hidden_feat, sample_weights,
                                     q_probs.reshape(N, 1), deg_part)
    zeros = jnp.zeros((ZR, D), jnp.float32)
    part = _sc_edges(hprime, src, dst,
                     hu.reshape(N), hv.reshape(N), zeros)
    return _tc_final(part, nd, W, b.reshape(1, D))


# speedup vs baseline: 44.4842x; 1.7400x over previous
import functools

import jax
import jax.numpy as jnp
from jax import lax
from jax.experimental import pallas as pl
from jax.experimental.pallas import tpu as pltpu
from jax.experimental.pallas import tpu_sc as plsc

N = 10000
E = 320000
D = 128

NC = 2
NS = 16
NW = NC * NS
EPW = E // NW
C = 80
NCHUNK = EPW // C
OWN = 1000
ZR = 250

_mesh = plsc.VectorSubcoreMesh(
    core_axis_name="c", subcore_axis_name="s", num_cores=NC, num_subcores=NS)
_sc_params = pltpu.CompilerParams(needs_layout_passes=False,
                                  use_tc_tiling_on_sc=False)
_sc_params_deg = _sc_params


@functools.partial(
    pl.kernel,
    out_type=jax.ShapeDtypeStruct((NC * N, 16), jnp.float32),
    mesh=_mesh,
    scratch_types=[
        pltpu.VMEM((C,), jnp.int32),
        pltpu.VMEM((C, 16), jnp.float32),
        pltpu.VMEM((ZR, 16), jnp.float32),
        pltpu.VMEM_SHARED((N, 16), jnp.float32),
    ],
    compiler_params=_sc_params_deg,
)
def _sc_degree(dst_hbm, deg_out, didx_v, ones_v, zbuf_v, acc_sh):
    cid = lax.axis_index("c")
    sid = lax.axis_index("s")
    wid = sid * NC + cid

    def ones_body(r, _):
        ones_v[r, :] = jnp.ones((16,), jnp.float32)
        return 0
    lax.fori_loop(0, C, ones_body, 0)

    def zfill_body(r, _):
        zbuf_v[r, :] = jnp.zeros((16,), jnp.float32)
        return 0
    lax.fori_loop(0, ZR, zfill_body, 0)

    @pl.when(sid < N // OWN)
    def _():
        def zero_body(k, _):
            pltpu.sync_copy(zbuf_v,
                            acc_sh.at[pl.ds(sid * OWN + k * ZR, ZR)])
            return 0
        lax.fori_loop(0, OWN // ZR, zero_body, 0)
    plsc.subcore_barrier()

    def chunk_body(i, _):
        base = wid * EPW + i * C
        pltpu.sync_copy(dst_hbm.at[pl.ds(base, C)], didx_v)
        pltpu.sync_copy(ones_v, acc_sh.at[didx_v], add=True)
        return 0
    lax.fori_loop(0, NCHUNK, chunk_body, 0)
    plsc.subcore_barrier()

    @pl.when(sid < N // OWN)
    def _():
        pltpu.sync_copy(acc_sh.at[pl.ds(sid * OWN, OWN)],
                        deg_out.at[pl.ds(cid * N + sid * OWN, OWN)])


def _tc_scalars_body(nf_ref, hid_ref, sw_ref, q_ref, degp_ref,
                     hu_ref, hv_ref, nd_ref, hp_ref):
    deg = degp_ref[0:N, 0:1] + degp_ref[N:2 * N, 0:1]
    nd = lax.rsqrt(deg + 1.0)
    nd_ref[...] = nd
    c = nd / (q_ref[...] * float(E))
    hp_ref[...] = hid_ref[...] * c
    nf = nf_ref[...]
    hu_ref[...] = jnp.dot(nf, sw_ref[:, 0:1],
                          preferred_element_type=jnp.float32)
    hv_ref[...] = jnp.dot(nf, sw_ref[:, 1:2],
                          preferred_element_type=jnp.float32)


def _tc_scalars(node_feat, hidden_feat, sample_weights, q2, deg_part):
    out = jax.ShapeDtypeStruct((N, 1), jnp.float32)
    return pl.pallas_call(
        _tc_scalars_body,
        out_shape=(out, out, out, jax.ShapeDtypeStruct((N, D), jnp.float32)),
    )(node_feat, hidden_feat, sample_weights, q2, deg_part)


@functools.partial(
    pl.kernel,
    out_type=jax.ShapeDtypeStruct((NC * N, D), jnp.float32),
    mesh=_mesh,
    scratch_types=[
        pltpu.VMEM((2, C), jnp.int32),
        pltpu.VMEM((2, C), jnp.int32),
        pltpu.VMEM((2, C), jnp.int32),
        pltpu.VMEM((2, C), jnp.int32),
        pltpu.VMEM((C, D), jnp.float32),
        pltpu.VMEM((C, D), jnp.float32),
        pltpu.VMEM((N,), jnp.float32),
        pltpu.VMEM((N,), jnp.float32),
        pltpu.SemaphoreType.DMA,
        pltpu.SemaphoreType.DMA,
        pltpu.SemaphoreType.DMA,
        pltpu.SemaphoreType.DMA,
        pltpu.VMEM_SHARED((N, D), jnp.float32),
    ],
    compiler_params=_sc_params,
)
def _sc_edges(hp_hbm, src_hbm, dst_hbm, hu_hbm, hv_hbm, zeros_hbm, part_out,
              sxA, dxA, sxB, dxB, rows0, rows1, hu_v, hv_v,
              ipsA, ipsB, gs0, gs1, acc_sh):
    cid = lax.axis_index("c")
    sid = lax.axis_index("s")
    wid = sid * NC + cid

    pltpu.sync_copy(hu_hbm, hu_v)
    pltpu.sync_copy(hv_hbm, hv_v)

    @pl.when(sid < N // OWN)
    def _():
        def zero_body(k, _):
            pltpu.sync_copy(zeros_hbm,
                            acc_sh.at[pl.ds(sid * OWN + k * ZR, ZR)])
            return 0
        lax.fori_loop(0, OWN // ZR, zero_body, 0)
    plsc.subcore_barrier()

    def issue_pair(p, sx, dx, sem):
        row = wid * NCHUNK + 2 * p
        pltpu.async_copy(src_hbm.at[pl.ds(row, 2)], sx, sem)
        pltpu.async_copy(dst_hbm.at[pl.ds(row, 2)], dx, sem)

    def wait_pair(sx, dx, sem):
        pltpu.make_async_copy(src_hbm.at[pl.ds(0, 2)], sx, sem).wait()
        pltpu.make_async_copy(dst_hbm.at[pl.ds(0, 2)], dx, sem).wait()

    def issue_gather(sx, j, rows, sem):
        pltpu.async_copy(hp_hbm.at[sx.at[j]], rows, sem)

    def wait_gather(sx, j, rows, sem):
        pltpu.make_async_copy(hp_hbm.at[sx.at[j]], rows, sem).wait()

    def compute_scatter(sx, dx, j, rows):
        def grp_body(g, _):
            s16 = pl.ds(g * 16, 16)
            si = sx[j, s16]
            di = dx[j, s16]
            hu16 = plsc.load_gather(hu_v, [si])
            hv16 = plsc.load_gather(hv_v, [di])
            s = jnp.maximum(hu16 + hv16, 0.0) + 0.1
            for e in range(16):
                a = s[e]
                row = g * 16 + e
                for jj in range(D // 16):
                    sl = pl.ds(jj * 16, 16)
                    rows[row, sl] = rows[row, sl] * a
            return 0
        lax.fori_loop(0, C // 16, grp_body, 0)
        pltpu.sync_copy(rows, acc_sh.at[dx.at[j]], add=True)

    NPAIR_LOOP = (NCHUNK - 1) // 4

    issue_pair(0, sxA, dxA, ipsA)
    wait_pair(sxA, dxA, ipsA)
    issue_gather(sxA, 0, rows0, gs0)
    issue_pair(1, sxB, dxB, ipsB)

    def pipe_body(k, _):
        issue_gather(sxA, 1, rows1, gs1)
        wait_gather(sxA, 0, rows0, gs0)
        compute_scatter(sxA, dxA, 0, rows0)
        wait_pair(sxB, dxB, ipsB)
        issue_gather(sxB, 0, rows0, gs0)
        wait_gather(sxA, 1, rows1, gs1)
        compute_scatter(sxA, dxA, 1, rows1)
        issue_pair(2 * k + 2, sxA, dxA, ipsA)
        issue_gather(sxB, 1, rows1, gs1)
        wait_gather(sxB, 0, rows0, gs0)
        compute_scatter(sxB, dxB, 0, rows0)
        wait_pair(sxA, dxA, ipsA)
        issue_gather(sxA, 0, rows0, gs0)
        wait_gather(sxB, 1, rows1, gs1)
        compute_scatter(sxB, dxB, 1, rows1)
        issue_pair(2 * k + 3, sxB, dxB, ipsB)
        return 0
    lax.fori_loop(0, NPAIR_LOOP, pipe_body, 0)

    wait_gather(sxA, 0, rows0, gs0)
    compute_scatter(sxA, dxA, 0, rows0)
    wait_pair(sxB, dxB, ipsB)

    plsc.subcore_barrier()

    @pl.when(sid < N // OWN)
    def _():
        pltpu.sync_copy(acc_sh.at[pl.ds(sid * OWN, OWN)],
                        part_out.at[pl.ds(cid * N + sid * OWN, OWN)])


def _tc_final_body(part_ref, nd_ref, w_ref, b_ref, out_ref):
    neigh = (part_ref[0:N, :] + part_ref[N:2 * N, :]) * nd_ref[...]
    out_ref[...] = lax.dot_general(
        neigh, w_ref[...], (((1,), (1,)), ((), ())),
        preferred_element_type=jnp.float32) + b_ref[...]


def _tc_final(part, nd2, W, b2):
    return pl.pallas_call(
        _tc_final_body,
        out_shape=jax.ShapeDtypeStruct((N, D), jnp.float32),
    )(part, nd2, W, b2)


def kernel(hidden_feat, node_feat, edge_index, sample_weights, q_probs, W, b):
    pad = jnp.zeros((4 * C,), jnp.int32)
    src = jnp.concatenate([edge_index[0].astype(jnp.int32), pad])
    dst = jnp.concatenate([edge_index[1].astype(jnp.int32), pad])
    src2d = src.reshape(-1, C)
    dst2d = dst.reshape(-1, C)

    deg_part = _sc_degree(dst[0:E])
    hu, hv, nd, hprime = _tc_scalars(node_feat, hidden_feat, sample_weights,
                                     q_probs.reshape(N, 1), deg_part)
    zeros = jnp.zeros((ZR, D), jnp.float32)
    part = _sc_edges(hprime, src2d, dst2d,
                     hu.reshape(N), hv.reshape(N), zeros)
    return _tc_final(part, nd, W, b.reshape(1, D))
